# baseline (speedup 1.0000x reference)
import functools
import math

import jax
import jax.numpy as jnp
from jax import lax
from jax.experimental import pallas as pl
from jax.experimental.pallas import tpu as pltpu
from jax.experimental.pallas import tpu_sc as plsc

D_MODEL = 64
SCALE = math.sqrt(D_MODEL)
NC = 2
NS = 16
NW = NC * NS
SEQ = 200
BATCH = 4096
VOCAB = 1000000
VOCAB2 = VOCAB // 2
PB = 128
NBLK = BATCH // PB
NPANEL = SEQ * NBLK
WP = NPANEL // NW

NFULL = VOCAB // 128
CBIG = 246
CSMALL = 244
TAIL_START = NFULL * 128
TAIL_W = VOCAB - TAIL_START

_mesh = plsc.VectorSubcoreMesh(core_axis_name="c", subcore_axis_name="s")
_params = pltpu.CompilerParams(
    use_tc_tiling_on_sc=True,
    needs_layout_passes=False,
    disable_bounds_checks=True,
)


@functools.partial(
    pl.kernel,
    out_type=jax.ShapeDtypeStruct((VOCAB2, 128), jnp.float32),
    mesh=_mesh,
    scratch_types=[
        pltpu.VMEM((2, D_MODEL, PB), jnp.float32),
        pltpu.VMEM((2, D_MODEL, PB), jnp.float32),
        pltpu.SemaphoreType.DMA,
        pltpu.SemaphoreType.DMA,
    ],
    compiler_params=_params,
)
def _relayout(lutT_hbm, tail_hbm, out_hbm, in_v, tr_v, gsem, wsem):
    wid = lax.axis_index("s") * NC + lax.axis_index("c")
    nc = jnp.where(wid < 2, CBIG, CSMALL)
    base = jnp.where(wid < 2, wid * CBIG, 2 * CBIG + (wid - 2) * CSMALL)
    iota16 = lax.iota(jnp.int32, 16)

    def fire(c, buf):
        pltpu.async_copy(
            lutT_hbm.at[:, pl.ds(c * 128, 128)], in_v.at[buf], gsem
        )

    def transpose_block(in_b, tr_b, nt0):
        @plsc.parallel_loop(0, nt0, unroll=2)
        def _(t0):
            tvec = t0 * 16 + iota16
            rowv = t0 * 8 + lax.shift_right_logical(iota16, 1)
            halfv = jnp.bitwise_and(iota16, 1) * 64
            for d in range(16):
                diag = jnp.bitwise_and(iota16 + d, 15)
                for f0 in range(0, D_MODEL, 16):
                    fvec = f0 + diag
                    v = plsc.load_gather(in_b, [fvec, tvec])
                    plsc.store_scatter(tr_b, [rowv, halfv + fvec], v)

    fire(base, 0)

    @pl.loop(0, CBIG, step=2)
    def _(ll):
        for b in range(2):
            lc = ll + b

            @pl.when(lc < nc)
            def _():
                c = base + lc
                in_b = in_v.at[b]
                tr_b = tr_v.at[b]

                @pl.when(lc > 0)
                def _():
                    pltpu.make_async_copy(
                        tr_v.at[1 - b], out_hbm.at[pl.ds(0, 64), :], wsem
                    ).wait()

                @pl.when(lc + 1 < nc)
                def _():
                    fire(c + 1, 1 - b)

                pltpu.make_async_copy(
                    lutT_hbm.at[:, pl.ds(0, 128)], in_b, gsem
                ).wait()

                transpose_block(in_b, tr_b, PB // 16)
                pltpu.async_copy(
                    tr_b, out_hbm.at[pl.ds(c * 64, 64), :], wsem
                )

    pltpu.make_async_copy(
        tr_v.at[1], out_hbm.at[pl.ds(0, 64), :], wsem
    ).wait()

    @pl.when(wid == NW - 1)
    def _():
        pltpu.sync_copy(tail_hbm, in_v.at[0].at[pl.ds(0, TAIL_W // 2), :])
        pltpu.sync_copy(
            in_v.at[0].at[pl.ds(0, TAIL_W // 2), :],
            out_hbm.at[pl.ds(TAIL_START // 2, TAIL_W // 2), :],
        )


@functools.partial(
    pl.kernel,
    out_type=jax.ShapeDtypeStruct((SEQ, D_MODEL, BATCH), jnp.float32),
    mesh=_mesh,
    scratch_types=[
        pltpu.VMEM((2, PB), jnp.int32),
        pltpu.VMEM((2, PB), jnp.int32),
        pltpu.VMEM((2, PB), jnp.int32),
        pltpu.VMEM((2, PB, 128), jnp.float32),
        pltpu.VMEM((2, D_MODEL, PB), jnp.float32),
        pltpu.SemaphoreType.DMA,
        pltpu.SemaphoreType.DMA,
    ],
    compiler_params=_params,
)
def _emb_lookup(xt_hbm, lut_hbm, out_hbm, idx_v, row_v, half_v, rows_v, tr_v,
                gsem, wsem):
    wid = lax.axis_index("s") * NC + lax.axis_index("c")
    pbase = wid * WP
    iota16 = lax.iota(jnp.int32, 16)

    def fire(p, buf):
        s = p // NBLK
        blk = p % NBLK
        pltpu.sync_copy(xt_hbm.at[s, pl.ds(blk * PB, PB)], idx_v.at[buf])
        for k in range(PB // 16):
            iv = idx_v[buf, pl.ds(k * 16, 16)]
            row_v[buf, pl.ds(k * 16, 16)] = lax.shift_right_logical(iv, 1)
            half_v[buf, pl.ds(k * 16, 16)] = lax.shift_left(
                jnp.bitwise_and(iv, 1), 6
            )
        pltpu.async_copy(lut_hbm.at[row_v.at[buf]], rows_v.at[buf], gsem)

    fire(pbase, 0)

    @pl.loop(0, WP, step=2)
    def _(pp):
        for b in range(2):
            lp = pp + b
            p = pbase + lp
            rows_b = rows_v.at[b]
            tr_b = tr_v.at[b]

            @pl.when(lp > 0)
            def _():
                pltpu.make_async_copy(
                    tr_v.at[1 - b], out_hbm.at[0, :, pl.ds(0, PB)], wsem
                ).wait()

            @pl.when(lp + 1 < WP)
            def _():
                fire(p + 1, 1 - b)

            pltpu.make_async_copy(
                lut_hbm.at[pl.ds(0, PB)], rows_b, gsem
            ).wait()

            @plsc.parallel_loop(0, PB // 16, unroll=2)
            def _(r0):
                rvec = r0 * 16 + iota16
                hv = half_v[b, pl.ds(r0 * 16, 16)]
                for d in range(16):
                    diag = jnp.bitwise_and(iota16 + d, 15)
                    for f0 in range(0, D_MODEL, 16):
                        jvec = diag + f0
                        v = plsc.load_gather(rows_b, [rvec, hv + jvec])
                        plsc.store_scatter(tr_b, [jvec, rvec], v * SCALE)

            s = p // NBLK
            blk = p % NBLK
            pltpu.async_copy(
                tr_b, out_hbm.at[s, :, pl.ds(blk * PB, PB)], wsem
            )

    pltpu.make_async_copy(
        tr_v.at[1], out_hbm.at[0, :, pl.ds(0, PB)], wsem
    ).wait()


def kernel(x, lut):
    tail = lut[TAIL_START:].reshape(TAIL_W // 2, 128)
    lut2 = _relayout(lut.T, tail)
    xt = x.astype(jnp.int32).T
    out_phys = _emb_lookup(xt, lut2)
    return jnp.transpose(out_phys, (2, 0, 1))


# speedup vs baseline: 1.3235x; 1.3235x over previous
import functools
import math

import jax
import jax.numpy as jnp
from jax import lax
from jax.experimental import pallas as pl
from jax.experimental.pallas import tpu as pltpu
from jax.experimental.pallas import tpu_sc as plsc

D_MODEL = 64
SCALE = math.sqrt(D_MODEL)
NC = 2
NS = 16
NW = NC * NS
SEQ = 200
BATCH = 4096
VOCAB = 1000000
VOCAB2 = VOCAB // 2
PB = 128
NBLK = BATCH // PB
NPANEL = SEQ * NBLK
WP = NPANEL // NW
XROWS = 7

WB = 256
NBLOCK = VOCAB // WB
CBIG = 123
CSMALL = 122
TAIL_START = NBLOCK * WB
TAIL_W = VOCAB - TAIL_START

_mesh = plsc.VectorSubcoreMesh(core_axis_name="c", subcore_axis_name="s")
_params = pltpu.CompilerParams(
    use_tc_tiling_on_sc=True,
    needs_layout_passes=False,
    disable_bounds_checks=True,
)


@functools.partial(
    pl.kernel,
    out_type=jax.ShapeDtypeStruct((VOCAB2, 128), jnp.float32),
    mesh=_mesh,
    scratch_types=[
        pltpu.VMEM((3, D_MODEL, WB), jnp.float32),
        pltpu.VMEM((3, WB // 2, 128), jnp.float32),
        pltpu.SemaphoreType.DMA,
        pltpu.SemaphoreType.DMA,
    ],
    compiler_params=_params,
)
def _relayout(lutT_hbm, tail_hbm, out_hbm, in_v, tr_v, gsem, wsem):
    wid = lax.axis_index("s") * NC + lax.axis_index("c")
    nc = jnp.where(wid < 2, CBIG, CSMALL)
    base = jnp.where(wid < 2, wid * CBIG, 2 * CBIG + (wid - 2) * CSMALL)
    iota16 = lax.iota(jnp.int32, 16)

    def fire(c, buf):
        pltpu.async_copy(
            lutT_hbm.at[:, pl.ds(c * WB, WB)], in_v.at[buf], gsem
        )

    def drain_g(buf):
        pltpu.make_async_copy(
            lutT_hbm.at[:, pl.ds(0, WB)], in_v.at[buf], gsem
        ).wait()

    def drain_w():
        pltpu.make_async_copy(
            tr_v.at[0], out_hbm.at[pl.ds(0, WB // 2), :], wsem
        ).wait()

    def transpose_block(in_b, tr_b, nt0):
        @plsc.parallel_loop(0, nt0, unroll=2)
        def _(t0):
            tvec = t0 * 16 + iota16
            rowv = t0 * 8 + lax.shift_right_logical(iota16, 1)
            halfv = jnp.bitwise_and(iota16, 1) * 64
            for d in range(16):
                diag = jnp.bitwise_and(iota16 + d, 15)
                for f0 in range(0, D_MODEL, 16):
                    fvec = f0 + diag
                    v = plsc.load_gather(in_b, [fvec, tvec])
                    plsc.store_scatter(tr_b, [rowv, halfv + fvec], v)

    fire(base, 0)
    fire(base + 1, 1)

    @pl.loop(0, CBIG, step=3)
    def _(ll):
        for b3 in range(3):
            lc = ll + b3
            buf = b3

            @pl.when(lc < nc)
            def _():
                c = base + lc
                in_b = in_v.at[buf]
                tr_b = tr_v.at[buf]

                @pl.when(lc >= 2)
                def _():
                    drain_w()

                @pl.when(lc + 2 < nc)
                def _():
                    fire(c + 2, (lc + 2) % 3)

                drain_g(buf)
                transpose_block(in_b, tr_b, WB // 16)
                pltpu.async_copy(
                    tr_b, out_hbm.at[pl.ds(c * (WB // 2), WB // 2), :], wsem
                )

    drain_w()
    drain_w()

    @pl.when(wid == NW - 1)
    def _():
        pltpu.sync_copy(tail_hbm, tr_v.at[0].at[pl.ds(0, TAIL_W // 2), :])
        pltpu.sync_copy(
            tr_v.at[0].at[pl.ds(0, TAIL_W // 2), :],
            out_hbm.at[pl.ds(TAIL_START // 2, TAIL_W // 2), :],
        )


@functools.partial(
    pl.kernel,
    out_type=jax.ShapeDtypeStruct((SEQ, D_MODEL, BATCH), jnp.float32),
    mesh=_mesh,
    scratch_types=[
        pltpu.VMEM((XROWS * BATCH,), jnp.int32),
        pltpu.VMEM((3, PB), jnp.int32),
        pltpu.VMEM((3, PB), jnp.int32),
        pltpu.VMEM((3, PB, 128), jnp.float32),
        pltpu.VMEM((3, D_MODEL, PB), jnp.float32),
        pltpu.SemaphoreType.DMA,
        pltpu.SemaphoreType.DMA,
    ],
    compiler_params=_params,
)
def _emb_lookup(xt_hbm, lut_hbm, out_hbm, idx_v, row_v, half_v, rows_v, tr_v,
                gsem, wsem):
    wid = lax.axis_index("s") * NC + lax.axis_index("c")
    pbase = wid * WP
    s0 = pbase // NBLK
    iota16 = lax.iota(jnp.int32, 16)

    for k in range(XROWS):
        pltpu.async_copy(
            xt_hbm.at[s0 + k], idx_v.at[pl.ds(k * BATCH, BATCH)], gsem
        )
    for k in range(XROWS):
        pltpu.make_async_copy(
            xt_hbm.at[0], idx_v.at[pl.ds(k * BATCH, BATCH)], gsem
        ).wait()

    def fire(lp, buf):
        p = pbase + lp
        off = (p // NBLK - s0) * BATCH + (p % NBLK) * PB
        for k in range(PB // 16):
            iv = idx_v[pl.ds(off + k * 16, 16)]
            row_v[buf, pl.ds(k * 16, 16)] = lax.shift_right_logical(iv, 1)
            half_v[buf, pl.ds(k * 16, 16)] = lax.shift_left(
                jnp.bitwise_and(iv, 1), 6
            )
        pltpu.async_copy(lut_hbm.at[row_v.at[buf]], rows_v.at[buf], gsem)

    def drain_g(buf):
        pltpu.make_async_copy(
            lut_hbm.at[pl.ds(0, PB)], rows_v.at[buf], gsem
        ).wait()

    def drain_w():
        pltpu.make_async_copy(
            tr_v.at[0], out_hbm.at[0, :, pl.ds(0, PB)], wsem
        ).wait()

    fire(0, 0)
    fire(1, 1)

    @pl.loop(0, WP, step=3)
    def _(pp):
        for b3 in range(3):
            lp = pp + b3
            buf = b3

            @pl.when(lp < WP)
            def _():
                p = pbase + lp
                rows_b = rows_v.at[buf]
                tr_b = tr_v.at[buf]

                @pl.when(lp >= 2)
                def _():
                    drain_w()

                @pl.when(lp + 2 < WP)
                def _():
                    fire(lp + 2, (lp + 2) % 3)

                drain_g(buf)

                @plsc.parallel_loop(0, PB // 16, unroll=2)
                def _(r0):
                    rvec = r0 * 16 + iota16
                    hv = half_v[buf, pl.ds(r0 * 16, 16)]
                    for d in range(16):
                        diag = jnp.bitwise_and(iota16 + d, 15)
                        for f0 in range(0, D_MODEL, 16):
                            jvec = diag + f0
                            v = plsc.load_gather(rows_b, [rvec, hv + jvec])
                            plsc.store_scatter(tr_b, [jvec, rvec], v * SCALE)

                s = p // NBLK
                blk = p % NBLK
                pltpu.async_copy(
                    tr_b, out_hbm.at[s, :, pl.ds(blk * PB, PB)], wsem
                )

    drain_w()
    drain_w()


def kernel(x, lut):
    tail = lut[TAIL_START:].reshape(TAIL_W // 2, 128)
    lut2 = _relayout(lut.T, tail)
    xt = x.astype(jnp.int32).T
    out_phys = _emb_lookup(xt, lut2)
    return jnp.transpose(out_phys, (2, 0, 1))


# speedup vs baseline: 1.3269x; 1.0025x over previous
import functools
import math

import jax
import jax.numpy as jnp
from jax import lax
from jax.experimental import pallas as pl
from jax.experimental.pallas import tpu as pltpu
from jax.experimental.pallas import tpu_sc as plsc

D_MODEL = 64
SCALE = math.sqrt(D_MODEL)
NC = 2
NS = 16
NW = NC * NS
SEQ = 200
BATCH = 4096
VOCAB = 1000000
VOCAB2 = VOCAB // 2
PB = 128
NBLK = BATCH // PB
NPANEL = SEQ * NBLK
WP = NPANEL // NW
XROWS = 7

WB = 256
NBLOCK = VOCAB // WB
CBIG = 123
CSMALL = 122
TAIL_START = NBLOCK * WB
TAIL_W = VOCAB - TAIL_START

_mesh = plsc.VectorSubcoreMesh(core_axis_name="c", subcore_axis_name="s")
_params = pltpu.CompilerParams(
    use_tc_tiling_on_sc=True,
    needs_layout_passes=False,
    disable_bounds_checks=True,
)


@functools.partial(
    pl.kernel,
    out_type=jax.ShapeDtypeStruct((VOCAB2, 128), jnp.float32),
    mesh=_mesh,
    scratch_types=[
        pltpu.VMEM((3, D_MODEL, WB), jnp.float32),
        pltpu.VMEM((3, WB // 2, 128), jnp.float32),
        pltpu.SemaphoreType.DMA,
        pltpu.SemaphoreType.DMA,
    ],
    compiler_params=_params,
)
def _relayout(lutT_hbm, tail_hbm, out_hbm, in_v, tr_v, gsem, wsem):
    wid = lax.axis_index("s") * NC + lax.axis_index("c")
    nc = jnp.where(wid < 2, CBIG, CSMALL)
    base = jnp.where(wid < 2, wid * CBIG, 2 * CBIG + (wid - 2) * CSMALL)
    iota16 = lax.iota(jnp.int32, 16)

    def fire(c, buf):
        pltpu.async_copy(
            lutT_hbm.at[:, pl.ds(c * WB, WB)], in_v.at[buf], gsem
        )

    def drain_g(buf):
        pltpu.make_async_copy(
            lutT_hbm.at[:, pl.ds(0, WB)], in_v.at[buf], gsem
        ).wait()

    def drain_w():
        pltpu.make_async_copy(
            tr_v.at[0], out_hbm.at[pl.ds(0, WB // 2), :], wsem
        ).wait()

    def transpose_block(in_b, tr_b, nt0):
        @plsc.parallel_loop(0, nt0, unroll=2)
        def _(t0):
            tvec = t0 * 16 + iota16
            rowv = t0 * 8 + lax.shift_right_logical(iota16, 1)
            halfv = jnp.bitwise_and(iota16, 1) * 64
            for d in range(16):
                diag = jnp.bitwise_and(iota16 + d, 15)
                for f0 in range(0, D_MODEL, 16):
                    fvec = f0 + diag
                    v = plsc.load_gather(in_b, [fvec, tvec])
                    plsc.store_scatter(tr_b, [rowv, halfv + fvec], v)

    fire(base, 0)
    fire(base + 1, 1)

    @pl.loop(0, CBIG, step=3)
    def _(ll):
        for b3 in range(3):
            lc = ll + b3
            buf = b3

            @pl.when(lc < nc)
            def _():
                c = base + lc
                in_b = in_v.at[buf]
                tr_b = tr_v.at[buf]

                @pl.when(lc >= 2)
                def _():
                    drain_w()

                @pl.when(lc + 2 < nc)
                def _():
                    fire(c + 2, (lc + 2) % 3)

                drain_g(buf)
                transpose_block(in_b, tr_b, WB // 16)
                pltpu.async_copy(
                    tr_b, out_hbm.at[pl.ds(c * (WB // 2), WB // 2), :], wsem
                )

    drain_w()
    drain_w()

    @pl.when(wid == NW - 1)
    def _():
        pltpu.sync_copy(tail_hbm, tr_v.at[0].at[pl.ds(0, TAIL_W // 2), :])
        pltpu.sync_copy(
            tr_v.at[0].at[pl.ds(0, TAIL_W // 2), :],
            out_hbm.at[pl.ds(TAIL_START // 2, TAIL_W // 2), :],
        )


@functools.partial(
    pl.kernel,
    out_type=jax.ShapeDtypeStruct((SEQ, D_MODEL, BATCH), jnp.float32),
    mesh=_mesh,
    scratch_types=[
        pltpu.VMEM((XROWS * BATCH,), jnp.int32),
        pltpu.VMEM((3, PB), jnp.int32),
        pltpu.VMEM((3, PB), jnp.int32),
        pltpu.VMEM((3, PB, 128), jnp.float32),
        pltpu.VMEM((3, D_MODEL, PB), jnp.float32),
        pltpu.SemaphoreType.DMA,
        pltpu.SemaphoreType.DMA,
    ],
    compiler_params=_params,
)
def _emb_lookup(xt_hbm, lut_hbm, out_hbm, idx_v, row_v, half_v, rows_v, tr_v,
                gsem, wsem):
    wid = lax.axis_index("s") * NC + lax.axis_index("c")
    pbase = wid * WP
    s0 = pbase // NBLK
    iota16 = lax.iota(jnp.int32, 16)

    for k in range(XROWS):
        pltpu.async_copy(
            xt_hbm.at[s0 + k], idx_v.at[pl.ds(k * BATCH, BATCH)], gsem
        )
    for k in range(XROWS):
        pltpu.make_async_copy(
            xt_hbm.at[0], idx_v.at[pl.ds(k * BATCH, BATCH)], gsem
        ).wait()

    def fire(lp, buf):
        p = pbase + lp
        off = (p // NBLK - s0) * BATCH + (p % NBLK) * PB
        for k in range(PB // 16):
            iv = idx_v[pl.ds(off + k * 16, 16)]
            row_v[buf, pl.ds(k * 16, 16)] = lax.shift_right_logical(iv, 1)
            half_v[buf, pl.ds(k * 16, 16)] = lax.shift_left(
                jnp.bitwise_and(iv, 1), 6
            )
        for h in range(2):
            pltpu.async_copy(
                lut_hbm.at[row_v.at[buf].at[pl.ds(h * 64, 64)]],
                rows_v.at[buf].at[pl.ds(h * 64, 64)],
                gsem,
            )

    def drain_g(buf):
        pltpu.make_async_copy(
            lut_hbm.at[pl.ds(0, PB)], rows_v.at[buf], gsem
        ).wait()

    def drain_w():
        pltpu.make_async_copy(
            tr_v.at[0], out_hbm.at[0, :, pl.ds(0, PB)], wsem
        ).wait()

    fire(0, 0)
    fire(1, 1)

    @pl.loop(0, WP, step=3)
    def _(pp):
        for b3 in range(3):
            lp = pp + b3
            buf = b3

            @pl.when(lp < WP)
            def _():
                p = pbase + lp
                rows_b = rows_v.at[buf]
                tr_b = tr_v.at[buf]

                @pl.when(lp >= 2)
                def _():
                    drain_w()

                @pl.when(lp + 2 < WP)
                def _():
                    fire(lp + 2, (lp + 2) % 3)

                drain_g(buf)

                @plsc.parallel_loop(0, PB // 16, unroll=2)
                def _(r0):
                    rvec = r0 * 16 + iota16
                    hv = half_v[buf, pl.ds(r0 * 16, 16)]
                    for d in range(16):
                        diag = jnp.bitwise_and(iota16 + d, 15)
                        for f0 in range(0, D_MODEL, 16):
                            jvec = diag + f0
                            v = plsc.load_gather(rows_b, [rvec, hv + jvec])
                            plsc.store_scatter(tr_b, [jvec, rvec], v * SCALE)

                s = p // NBLK
                blk = p % NBLK
                pltpu.async_copy(
                    tr_b, out_hbm.at[s, :, pl.ds(blk * PB, PB)], wsem
                )

    drain_w()
    drain_w()


def kernel(x, lut):
    tail = lut[TAIL_START:].reshape(TAIL_W // 2, 128)
    lut2 = _relayout(lut.T, tail)
    xt = x.astype(jnp.int32).T
    out_phys = _emb_lookup(xt, lut2)
    return jnp.transpose(out_phys, (2, 0, 1))
